# baseline (speedup 1.0000x reference)
import functools

import jax
import jax.numpy as jnp
from jax.experimental import pallas as pl
from jax.experimental.pallas import tpu as pltpu
from jax.experimental.pallas import tpu_sc as plsc


_GATHER_WINDOW = 128
_ROW_BLK = 1024


_ROWS_PER_STEP = 64


def _sc_gather(flat_tables, idx2d, emb):
    batch, nf = idx2d.shape
    rows = _ROWS_PER_STEP
    flat_per_step = rows * nf
    n_windows = flat_per_step // _GATHER_WINDOW
    assert flat_per_step % _GATHER_WINDOW == 0 and batch % rows == 0
    mesh = plsc.VectorSubcoreMesh(core_axis_name="core", subcore_axis_name="subcore")

    @functools.partial(
        pl.kernel,
        out_type=jax.ShapeDtypeStruct((batch, nf * emb), flat_tables.dtype),
        mesh=mesh,
        scratch_types=[
            pltpu.VMEM((flat_per_step,), jnp.int32),
            pltpu.VMEM((flat_per_step, emb), flat_tables.dtype),
            pltpu.SemaphoreType.DMA,
        ],
        compiler_params=pltpu.CompilerParams(use_tc_tiling_on_sc=False),
    )
    def gather_kernel(tab_hbm, idx_hbm, out_hbm, flat_ref, rows_ref, sem):
        def body(idx_vmem, out_vmem):
            @pl.loop(0, rows)
            def _(r):
                flat_ref[pl.ds(r * nf, 16)] = idx_vmem[r, pl.ds(0, 16)]
                flat_ref[pl.ds(r * nf + (nf - 16), 16)] = idx_vmem[r, pl.ds(nf - 16, 16)]

            copies = [
                pltpu.async_copy(
                    tab_hbm.at[flat_ref.at[pl.ds(w * _GATHER_WINDOW, _GATHER_WINDOW)]],
                    rows_ref.at[pl.ds(w * _GATHER_WINDOW, _GATHER_WINDOW)],
                    sem,
                )
                for w in range(n_windows)
            ]
            for c in copies:
                c.wait()

            @pl.loop(0, rows)
            def _(r):
                for c in range(nf):
                    out_vmem[r, pl.ds(c * emb, emb)] = rows_ref[r * nf + c, pl.ds(0, emb)]

        pltpu.emit_pipeline(
            body,
            grid=(batch // rows,),
            in_specs=[
                pl.BlockSpec((rows, nf), index_map=lambda i: (i, 0))
            ],
            out_specs=[
                pl.BlockSpec((rows, nf * emb), index_map=lambda i: (i, 0))
            ],
            core_axis_name=("core", "subcore"),
            dimension_semantics=(pltpu.PARALLEL,),
        )(idx_hbm, out_hbm)

    return gather_kernel(flat_tables, idx2d)


def _mlp_body(n_res, dense_ref, emb_ref, W1d_ref, W1e_ref, b1_ref,
              W2d_ref, W2e_ref, b2d_ref, b2e_ref, Wfd_ref, Wfe_ref, bf_ref,
              out_ref):
    f32 = jnp.float32
    r_d = dense_ref[...]
    r_e = emb_ref[...]
    for l in range(n_res):
        h = (jnp.dot(r_d, W1d_ref[l], preferred_element_type=f32)
             + jnp.dot(r_e, W1e_ref[l], preferred_element_type=f32)
             + b1_ref[l][None, :])
        h = jnp.maximum(h, 0.0)
        x_d = jnp.dot(h, W2d_ref[l], preferred_element_type=f32) + b2d_ref[l][None, :]
        x_e = jnp.dot(h, W2e_ref[l], preferred_element_type=f32) + b2e_ref[l][None, :]
        r_d = jnp.maximum(x_d + r_d, 0.0)
        r_e = jnp.maximum(x_e + r_e, 0.0)
    logit = (jnp.dot(r_d, Wfd_ref[...], preferred_element_type=f32)
             + jnp.dot(r_e, Wfe_ref[...], preferred_element_type=f32)
             + bf_ref[0, 0])
    out_ref[...] = jax.nn.sigmoid(logit)


def _tc_mlp(dense, emb, W1d, W1e, b1, W2d, W2e, b2d, b2e, Wfd, Wfe, bf2):
    batch, d_dense = dense.shape
    d_emb = emb.shape[1]
    n_res, _, hidden = W1e.shape
    blk = _ROW_BLK
    grid = (batch // blk,)

    def row_map(i):
        return (i, 0)

    def const2(i):
        return (0, 0)

    def const3(i):
        return (0, 0, 0)

    return pl.pallas_call(
        functools.partial(_mlp_body, n_res),
        grid=grid,
        in_specs=[
            pl.BlockSpec((blk, d_dense), row_map),
            pl.BlockSpec((blk, d_emb), row_map),
            pl.BlockSpec((n_res, d_dense, hidden), const3),
            pl.BlockSpec((n_res, d_emb, hidden), const3),
            pl.BlockSpec((n_res, hidden), const2),
            pl.BlockSpec((n_res, hidden, d_dense), const3),
            pl.BlockSpec((n_res, hidden, d_emb), const3),
            pl.BlockSpec((n_res, d_dense), const2),
            pl.BlockSpec((n_res, d_emb), const2),
            pl.BlockSpec((d_dense, 1), const2),
            pl.BlockSpec((d_emb, 1), const2),
            pl.BlockSpec((1, 1), const2),
        ],
        out_specs=pl.BlockSpec((blk, 1), row_map),
        out_shape=jax.ShapeDtypeStruct((batch, 1), jnp.float32),
    )(dense, emb, W1d, W1e, b1, W2d, W2e, b2d, b2e, Wfd, Wfe, bf2)


def kernel(dense_inputs, sparse_inputs, tables, W1, b1, W2, b2, Wf, bf):
    batch, d_dense = dense_inputs.shape
    n_fields = sparse_inputs.shape[1]
    vocab, emb = tables.shape[1], tables.shape[2]
    num_indices = batch * n_fields

    offs = (jnp.arange(n_fields, dtype=jnp.int32) * vocab)[None, :]
    idx2d = sparse_inputs.astype(jnp.int32) + offs
    flat_tables = tables.reshape(n_fields * vocab, emb)

    emb_cat = _sc_gather(flat_tables, idx2d, emb)

    W1d, W1e = W1[:, :d_dense, :], W1[:, d_dense:, :]
    W2d, W2e = W2[:, :, :d_dense], W2[:, :, d_dense:]
    b2d, b2e = b2[:, :d_dense], b2[:, d_dense:]
    Wfd, Wfe = Wf[:d_dense, :], Wf[d_dense:, :]
    bf2 = bf.reshape(1, 1)

    return _tc_mlp(dense_inputs, emb_cat, W1d, W1e, b1,
                   W2d, W2e, b2d, b2e, Wfd, Wfe, bf2)


# speedup vs baseline: 3.7108x; 3.7108x over previous
import functools

import jax
import jax.numpy as jnp
from jax.experimental import pallas as pl
from jax.experimental.pallas import tpu as pltpu
from jax.experimental.pallas import tpu_sc as plsc


_GATHER_WINDOW = 128
_ROW_BLK = 2048
_ROWS_PER_STEP = 64


def _detile2_body(vb, emb, seg, sub, in_ref, out_ref):
    x = in_ref[0]
    f32 = jnp.float32
    eye = jnp.eye(128, dtype=f32)
    dn = (((0,), (0,)), ((), ()))
    for s in range(seg // sub):
        parts = []
        for k in range(8):
            c0 = k * seg + s * sub
            c1 = c0 + sub
            if c1 <= vb:
                parts.append(x[:, c0:c1])
            elif c0 < vb:
                parts.append(jnp.concatenate(
                    [x[:, c0:vb], jnp.zeros((emb, c1 - vb), x.dtype)], axis=1))
            else:
                parts.append(jnp.zeros((emb, sub), x.dtype))
        xs = jnp.concatenate(parts, axis=0)
        out_ref[s * sub:(s + 1) * sub, :] = jax.lax.dot_general(
            xs, eye, dn, preferred_element_type=f32)


def _tc_detile2(tables_t, nf, emb):
    _, _, vocab = tables_t.shape
    sub = -(-vocab // 32)
    sub = -(-sub // 8) * 8
    seg = sub * 4
    return pl.pallas_call(
        functools.partial(_detile2_body, vocab, emb, seg, sub),
        grid=(nf,),
        in_specs=[pl.BlockSpec((1, emb, vocab), lambda f: (f, 0, 0))],
        out_specs=pl.BlockSpec((seg, 128), lambda f: (f, 0)),
        out_shape=jax.ShapeDtypeStruct((nf * seg, 128), tables_t.dtype),
    )(tables_t)


def _sc_gather(flat_tables, idx2d, emb):
    batch, nf = idx2d.shape
    rows = _ROWS_PER_STEP
    flat_per_step = rows * nf
    n_windows = flat_per_step // _GATHER_WINDOW
    assert flat_per_step % _GATHER_WINDOW == 0 and batch % rows == 0
    mesh = plsc.VectorSubcoreMesh(core_axis_name="core", subcore_axis_name="subcore")

    @functools.partial(
        pl.kernel,
        out_type=jax.ShapeDtypeStruct((batch, nf * emb), flat_tables.dtype),
        mesh=mesh,
        scratch_types=[
            pltpu.VMEM((flat_per_step,), jnp.int32),
            pltpu.VMEM((flat_per_step, emb), flat_tables.dtype),
            pltpu.SemaphoreType.DMA,
        ],
        compiler_params=pltpu.CompilerParams(use_tc_tiling_on_sc=False),
    )
    def gather_kernel(tab_hbm, idx_hbm, out_hbm, flat_ref, rows_ref, sem):
        def body(idx_vmem, out_vmem):
            @pl.loop(0, rows)
            def _(r):
                flat_ref[pl.ds(r * nf, 16)] = idx_vmem[r, pl.ds(0, 16)]
                flat_ref[pl.ds(r * nf + (nf - 16), 16)] = idx_vmem[r, pl.ds(nf - 16, 16)]

            copies = [
                pltpu.async_copy(
                    tab_hbm.at[flat_ref.at[pl.ds(w * _GATHER_WINDOW, _GATHER_WINDOW)]],
                    rows_ref.at[pl.ds(w * _GATHER_WINDOW, _GATHER_WINDOW)],
                    sem,
                )
                for w in range(n_windows)
            ]
            for c in copies:
                c.wait()

            @pl.loop(0, rows)
            def _(r):
                for c in range(nf):
                    out_vmem[r, pl.ds(c * emb, emb)] = rows_ref[r * nf + c, pl.ds(0, emb)]

        pltpu.emit_pipeline(
            body,
            grid=(batch // rows,),
            in_specs=[
                pl.BlockSpec((rows, nf), index_map=lambda i: (i, 0))
            ],
            out_specs=[
                pl.BlockSpec((rows, nf * emb), index_map=lambda i: (i, 0))
            ],
            core_axis_name=("core", "subcore"),
            dimension_semantics=(pltpu.PARALLEL,),
        )(idx_hbm, out_hbm)

    return gather_kernel(flat_tables, idx2d)


def _mlp_body(n_res, dense_ref, emb_ref, W1d_ref, W1e_ref, b1_ref,
              W2d_ref, W2e_ref, b2d_ref, b2e_ref, Wfd_ref, Wfe_ref, bf_ref,
              out_ref):
    f32 = jnp.float32
    r_d = dense_ref[...]
    r_e = emb_ref[...]
    for l in range(n_res):
        h = (jnp.dot(r_d, W1d_ref[l], preferred_element_type=f32)
             + jnp.dot(r_e, W1e_ref[l], preferred_element_type=f32)
             + b1_ref[l][None, :])
        h = jnp.maximum(h, 0.0)
        x_d = jnp.dot(h, W2d_ref[l], preferred_element_type=f32) + b2d_ref[l][None, :]
        x_e = jnp.dot(h, W2e_ref[l], preferred_element_type=f32) + b2e_ref[l][None, :]
        r_d = jnp.maximum(x_d + r_d, 0.0)
        r_e = jnp.maximum(x_e + r_e, 0.0)
    logit = (jnp.dot(r_d, Wfd_ref[...], preferred_element_type=f32)
             + jnp.dot(r_e, Wfe_ref[...], preferred_element_type=f32)
             + bf_ref[0, 0])
    out_ref[...] = jax.nn.sigmoid(logit)


def _tc_mlp(dense, emb, W1d, W1e, b1, W2d, W2e, b2d, b2e, Wfd, Wfe, bf2):
    batch, d_dense = dense.shape
    d_emb = emb.shape[1]
    n_res, _, hidden = W1e.shape
    blk = _ROW_BLK
    grid = (batch // blk,)

    def row_map(i):
        return (i, 0)

    def const2(i):
        return (0, 0)

    def const3(i):
        return (0, 0, 0)

    return pl.pallas_call(
        functools.partial(_mlp_body, n_res),
        grid=grid,
        in_specs=[
            pl.BlockSpec((blk, d_dense), row_map),
            pl.BlockSpec((blk, d_emb), row_map),
            pl.BlockSpec((n_res, d_dense, hidden), const3),
            pl.BlockSpec((n_res, d_emb, hidden), const3),
            pl.BlockSpec((n_res, hidden), const2),
            pl.BlockSpec((n_res, hidden, d_dense), const3),
            pl.BlockSpec((n_res, hidden, d_emb), const3),
            pl.BlockSpec((n_res, d_dense), const2),
            pl.BlockSpec((n_res, d_emb), const2),
            pl.BlockSpec((d_dense, 1), const2),
            pl.BlockSpec((d_emb, 1), const2),
            pl.BlockSpec((1, 1), const2),
        ],
        out_specs=pl.BlockSpec((blk, 1), row_map),
        out_shape=jax.ShapeDtypeStruct((batch, 1), jnp.float32),
    )(dense, emb, W1d, W1e, b1, W2d, W2e, b2d, b2e, Wfd, Wfe, bf2)


def kernel(dense_inputs, sparse_inputs, tables, W1, b1, W2, b2, Wf, bf):
    batch, d_dense = dense_inputs.shape
    n_fields = sparse_inputs.shape[1]
    emb = tables.shape[2]

    tables_t = jnp.transpose(tables, (0, 2, 1))
    tab128 = _tc_detile2(tables_t, n_fields, emb)
    seg = tab128.shape[0] // n_fields
    tab_flat = tab128.reshape(tab128.shape[0] * 128 // emb, emb)

    sp = sparse_inputs.astype(jnp.int32)
    offs = (jnp.arange(n_fields, dtype=jnp.int32) * (seg * 8))[None, :]
    idx2d = offs + (sp % seg) * 8 + sp // seg

    emb_cat = _sc_gather(tab_flat, idx2d, emb)

    W1d, W1e = W1[:, :d_dense, :], W1[:, d_dense:, :]
    W2d, W2e = W2[:, :, :d_dense], W2[:, :, d_dense:]
    b2d, b2e = b2[:, :d_dense], b2[:, d_dense:]
    Wfd, Wfe = Wf[:d_dense, :], Wf[d_dense:, :]
    bf2 = bf.reshape(1, 1)

    return _tc_mlp(dense_inputs, emb_cat, W1d, W1e, b1,
                   W2d, W2e, b2d, b2e, Wfd, Wfe, bf2)
